# baseline (speedup 1.0000x reference)
import functools
import math

import jax
import jax.numpy as jnp
from jax.experimental import pallas as pl
from jax.experimental.pallas import tpu as pltpu

S = 2048
HIDDEN = 4096
E = 64
CAP = 64
BS = 256
NB = S // BS
EPS = float(jnp.finfo(jnp.float32).eps)


def _gate_stage(x_ref, w_ref, tri_ref,
                e1_ref, e2_ref, g1_ref, g2_ref, loc1_ref, loc2p_ref,
                cnt_ref, gsum_ref,
                carry1, carry2, gacc):
    b = pl.program_id(0)

    @pl.when(b == 0)
    def _init():
        carry1[...] = jnp.zeros_like(carry1)
        carry2[...] = jnp.zeros_like(carry2)
        gacc[...] = jnp.zeros_like(gacc)

    x = x_ref[...]
    w = w_ref[...]
    logits = jax.lax.dot_general(x, w, (((1,), (1,)), ((), ())),
                                 preferred_element_type=jnp.float32)
    m = jnp.max(logits, axis=1, keepdims=True)
    ex = jnp.exp(logits - m)
    gates = ex / jnp.sum(ex, axis=1, keepdims=True)

    eio = jax.lax.broadcasted_iota(jnp.int32, (BS, E), 1)
    g1 = jnp.max(gates, axis=1, keepdims=True)
    e1 = jnp.min(jnp.where(gates == g1, eio, E), axis=1, keepdims=True)
    hit1 = eio == e1
    gates_m = jnp.where(hit1, -1.0, gates)
    g2 = jnp.max(gates_m, axis=1, keepdims=True)
    e2 = jnp.min(jnp.where(gates_m == g2, eio, E), axis=1, keepdims=True)
    hit2 = eio == e2

    mask1 = hit1.astype(jnp.float32)
    mask2 = hit2.astype(jnp.float32)
    tri = tri_ref[...]
    cs1 = jax.lax.dot_general(tri, mask1, (((1,), (0,)), ((), ())),
                              preferred_element_type=jnp.float32)
    cs2 = jax.lax.dot_general(tri, mask2, (((1,), (0,)), ((), ())),
                              preferred_element_type=jnp.float32)
    c1 = carry1[0:1, :]
    c2 = carry2[0:1, :]
    loc1 = cs1 - 1.0 + c1
    loc2p = cs2 - 1.0 + c2

    e1_ref[...] = e1
    e2_ref[...] = e2
    g1_ref[...] = g1
    g2_ref[...] = g2
    loc1_ref[...] = jnp.sum(loc1 * mask1, axis=1, keepdims=True).astype(jnp.int32)
    loc2p_ref[...] = jnp.sum(loc2p * mask2, axis=1, keepdims=True).astype(jnp.int32)

    carry1[0:1, :] = c1 + cs1[BS - 1:BS, :]
    carry2[0:1, :] = c2 + cs2[BS - 1:BS, :]
    gacc[0:1, :] = gacc[0:1, :] + jnp.sum(gates, axis=0, keepdims=True)

    cnt_ref[...] = carry1[0:1, :].astype(jnp.int32)
    gsum_ref[...] = gacc[0:1, :]


def _dense_stage(e1_ref, e2_ref, g1_ref, g2_ref, loc1_ref, loc2p_ref,
                 cnt_ref, comb_ref, mask_ref):
    e1 = e1_ref[...]
    e2 = e2_ref[...]
    g1 = g1_ref[...]
    g2 = g2_ref[...]
    loc1 = loc1_ref[...]
    loc2p = loc2p_ref[...]
    cnt = cnt_ref[...]

    eio = jax.lax.broadcasted_iota(jnp.int32, (BS, E), 1)
    cnt2 = jnp.sum(jnp.where(eio == e2, cnt, 0), axis=1, keepdims=True)
    loc2 = loc2p + cnt2

    kept1 = loc1 < CAP
    kept2 = loc2 < CAP
    g1k = jnp.where(kept1, g1, 0.0)
    g2k = jnp.where(kept2, g2, 0.0)
    denom = jnp.maximum(g1k + g2k, EPS)
    c1 = g1k / denom
    c2 = g2k / denom
    idx1 = jnp.where(kept1, e1 * CAP + loc1, -1)
    idx2 = jnp.where(kept2, e2 * CAP + loc2, -1)

    j = jax.lax.broadcasted_iota(jnp.int32, (BS, E * CAP), 1)
    h1 = j == idx1
    h2 = j == idx2
    comb_ref[...] = jnp.where(h1, c1, 0.0) + jnp.where(h2, c2, 0.0)
    mask_ref[...] = h1 | h2


@jax.jit
def kernel(input, W):
    x = input.astype(jnp.float32)
    tri = jnp.tril(jnp.ones((BS, BS), jnp.float32))

    outs = pl.pallas_call(
        _gate_stage,
        grid=(NB,),
        in_specs=[
            pl.BlockSpec((BS, HIDDEN), lambda b: (b, 0)),
            pl.BlockSpec((E, HIDDEN), lambda b: (0, 0)),
            pl.BlockSpec((BS, BS), lambda b: (0, 0)),
        ],
        out_specs=[
            pl.BlockSpec((BS, 1), lambda b: (b, 0)),
            pl.BlockSpec((BS, 1), lambda b: (b, 0)),
            pl.BlockSpec((BS, 1), lambda b: (b, 0)),
            pl.BlockSpec((BS, 1), lambda b: (b, 0)),
            pl.BlockSpec((BS, 1), lambda b: (b, 0)),
            pl.BlockSpec((BS, 1), lambda b: (b, 0)),
            pl.BlockSpec((1, E), lambda b: (0, 0)),
            pl.BlockSpec((1, E), lambda b: (0, 0)),
        ],
        out_shape=[
            jax.ShapeDtypeStruct((S, 1), jnp.int32),
            jax.ShapeDtypeStruct((S, 1), jnp.int32),
            jax.ShapeDtypeStruct((S, 1), jnp.float32),
            jax.ShapeDtypeStruct((S, 1), jnp.float32),
            jax.ShapeDtypeStruct((S, 1), jnp.int32),
            jax.ShapeDtypeStruct((S, 1), jnp.int32),
            jax.ShapeDtypeStruct((1, E), jnp.int32),
            jax.ShapeDtypeStruct((1, E), jnp.float32),
        ],
        scratch_shapes=[
            pltpu.VMEM((8, E), jnp.float32),
            pltpu.VMEM((8, E), jnp.float32),
            pltpu.VMEM((8, E), jnp.float32),
        ],
        compiler_params=pltpu.CompilerParams(
            dimension_semantics=("arbitrary",)),
    )(x, W, tri)
    e1, e2, g1, g2, loc1, loc2p, cnt, gsum = outs

    comb, mask = pl.pallas_call(
        _dense_stage,
        grid=(NB,),
        in_specs=[
            pl.BlockSpec((BS, 1), lambda b: (b, 0)),
            pl.BlockSpec((BS, 1), lambda b: (b, 0)),
            pl.BlockSpec((BS, 1), lambda b: (b, 0)),
            pl.BlockSpec((BS, 1), lambda b: (b, 0)),
            pl.BlockSpec((BS, 1), lambda b: (b, 0)),
            pl.BlockSpec((BS, 1), lambda b: (b, 0)),
            pl.BlockSpec((1, E), lambda b: (0, 0)),
        ],
        out_specs=[
            pl.BlockSpec((BS, E * CAP), lambda b: (b, 0)),
            pl.BlockSpec((BS, E * CAP), lambda b: (b, 0)),
        ],
        out_shape=[
            jax.ShapeDtypeStruct((S, E * CAP), jnp.float32),
            jax.ShapeDtypeStruct((S, E * CAP), jnp.bool_),
        ],
        compiler_params=pltpu.CompilerParams(
            dimension_semantics=("arbitrary",)),
    )(e1, e2, g1, g2, loc1, loc2p, cnt)

    exp_counts = cnt.reshape(E)
    me = gsum.reshape(E) / S
    ce = exp_counts.astype(jnp.float32) / S
    l_aux = jnp.mean(me * ce) * E * E
    combine_weights = comb.reshape(S, E, CAP)
    dispatch_mask = mask.reshape(S, E, CAP)
    return (l_aux, combine_weights, dispatch_mask, exp_counts)
